# baseline (speedup 1.0000x reference)
import jax
import jax.numpy as jnp
from jax import lax
from jax.experimental import pallas as pl
from jax.experimental.pallas import tpu as pltpu
from jax.experimental.pallas import tpu_sc as plsc

H = 64
NV = H // 16
SEQ = 200
BLK = 128
NC = 2
NS = 16
NW = NC * NS
NBUF = 4


def _lane_sum(v):
  ps = plsc.cumsum(v)
  last = lax.iota(jnp.int32, 16) | 15
  return ps.at[last].get(mode="promise_in_bounds")


def _rsqrt(x):
  i = lax.bitcast_convert_type(x, jnp.int32)
  i = jnp.int32(0x5F3759DF) - (i >> 1)
  y = lax.bitcast_convert_type(i, jnp.float32)
  y = y * (1.5 - 0.5 * x * y * y)
  return y * (1.5 - 0.5 * x * y * y)


def _body(ids_hbm, table_hbm, pos_hbm, gam_hbm, bet_hbm, out_hbm,
          idx_all, rows, outs, pos_v, gam_v, bet_v, gsems, osems):
  bpw = ids_hbm.shape[0] // NW
  wid = lax.axis_index("s") * NC + lax.axis_index("c")
  base = wid * bpw

  pltpu.sync_copy(ids_hbm.at[pl.ds(base, bpw)], idx_all)
  pltpu.sync_copy(pos_hbm, pos_v)
  pltpu.sync_copy(gam_hbm, gam_v)
  pltpu.sync_copy(bet_hbm, bet_v)
  g = [gam_v[pl.ds(16 * j, 16)] for j in range(NV)]
  b = [bet_v[pl.ds(16 * j, 16)] for j in range(NV)]
  lanes = lax.iota(jnp.int32, 16)
  hidx = [(lanes + 16 * j) >> 3 for j in range(NV)]
  midx = [(lanes + 16 * j) & 7 for j in range(NV)]
  zidx = lanes & 0

  def oref(blk):
    return out_hbm.at[blk // 32, :, pl.ds(blk % 32, 1)]

  def compute_block(rv, ov, t, carry):
    p = [pos_v[t, pl.ds(16 * j, 16)] for j in range(NV)]

    @plsc.parallel_loop(0, BLK, unroll=2)
    def row_body(r):
      x = [rv[r, pl.ds(16 * j, 16)] + p[j] for j in range(NV)]
      s = (x[0] + x[1]) + (x[2] + x[3])
      q = (x[0] * x[0] + x[1] * x[1]) + (x[2] * x[2] + x[3] * x[3])
      mean = _lane_sum(s) * (1.0 / H)
      var = _lane_sum(q) * (1.0 / H) - mean * mean
      inv = _rsqrt(var + 1e-12)
      rsp = jnp.full((16,), r, dtype=jnp.int32)
      for j in range(NV):
        y = (x[j] - mean) * inv * g[j] + b[j]
        plsc.store_scatter(ov, [hidx[j], zidx, midx[j], rsp], y)
    return carry

  pltpu.async_copy(table_hbm.at[idx_all.at[0]], rows[0], gsems[0])
  pltpu.async_copy(table_hbm.at[idx_all.at[1]], rows[1], gsems[1])
  pltpu.async_copy(table_hbm.at[idx_all.at[2]], rows[2], gsems[2])

  def ring_body(it, carry):
    for bb in range(NBUF):
      c = it * NBUF + bb
      b2 = (bb + 3) % NBUF

      @pl.when(c + 3 < bpw)
      def _():
        pltpu.async_copy(
            table_hbm.at[idx_all.at[c + 3]], rows[b2], gsems[b2])

      pltpu.make_async_copy(
          table_hbm.at[idx_all.at[c]], rows[bb], gsems[bb]).wait()

      @pl.when(c >= NBUF)
      def _():
        pltpu.make_async_copy(outs[bb].at[:, :, :, pl.ds(0, BLK)],
                              oref(base + c - NBUF), osems[bb]).wait()

      compute_block(rows[bb], outs[bb], (base + c) // 32, 0)
      pltpu.async_copy(outs[bb].at[:, :, :, pl.ds(0, BLK)],
                       oref(base + c), osems[bb])
    return carry

  lax.fori_loop(0, bpw // NBUF, ring_body, 0)

  for bb in range(NBUF):
    pltpu.make_async_copy(
        outs[bb].at[:, :, :, pl.ds(0, BLK)],
        oref(base + bpw - NBUF + bb), osems[bb]).wait()


def kernel(input_ids, item_table, pos_table, ln_gamma, ln_beta):
  batch, seq = input_ids.shape
  nblk = batch * seq // BLK
  ids_t = input_ids.T.astype(jnp.int32).reshape(nblk, BLK)

  def body(ids_hbm, table_hbm, pos_hbm, gam_hbm, bet_hbm, out_hbm,
           idx_all, r0, r1, r2, r3, o0, o1, o2, o3, pos_v, gam_v, bet_v,
           g0, g1, g2, g3, s0, s1, s2, s3):
    _body(ids_hbm, table_hbm, pos_hbm, gam_hbm, bet_hbm, out_hbm,
          idx_all, [r0, r1, r2, r3], [o0, o1, o2, o3], pos_v, gam_v, bet_v,
          [g0, g1, g2, g3], [s0, s1, s2, s3])

  mesh = plsc.VectorSubcoreMesh(core_axis_name="c", subcore_axis_name="s")
  run = pl.kernel(
      body,
      mesh=mesh,
      compiler_params=pltpu.CompilerParams(
          use_tc_tiling_on_sc=False, needs_layout_passes=False),
      out_type=jax.ShapeDtypeStruct((SEQ, 8, batch // BLK, 8, BLK),
                                    jnp.float32),
      scratch_types=(
          [pltpu.VMEM((nblk // NW, BLK), jnp.int32)]
          + [pltpu.VMEM((BLK, H), jnp.float32) for _ in range(NBUF)]
          + [pltpu.VMEM((8, 1, 8, BLK + 1), jnp.float32) for _ in range(NBUF)]
          + [pltpu.VMEM((SEQ, H), jnp.float32),
             pltpu.VMEM((H,), jnp.float32),
             pltpu.VMEM((H,), jnp.float32)]
          + [pltpu.SemaphoreType.DMA for _ in range(2 * NBUF)]
      ),
  )
  out5 = run(ids_t, item_table, pos_table, ln_gamma, ln_beta)
  return out5.transpose(2, 4, 0, 1, 3).reshape(batch, seq, H)


# speedup vs baseline: 1.0712x; 1.0712x over previous
import jax
import jax.numpy as jnp
from jax import lax
from jax.experimental import pallas as pl
from jax.experimental.pallas import tpu as pltpu
from jax.experimental.pallas import tpu_sc as plsc

H = 64
NV = H // 16
SEQ = 200
BLK = 128
NC = 2
NS = 16
NW = NC * NS
NBUF = 4


def _lane_sum(v):
  ps = plsc.cumsum(v)
  last = lax.iota(jnp.int32, 16) | 15
  return ps.at[last].get(mode="promise_in_bounds")


def _rsqrt(x):
  i = lax.bitcast_convert_type(x, jnp.int32)
  i = jnp.int32(0x5F3759DF) - (i >> 1)
  y = lax.bitcast_convert_type(i, jnp.float32)
  y = y * (1.5 - 0.5 * x * y * y)
  return y * (1.5 - 0.5 * x * y * y)


def _body(ids_hbm, table_hbm, pos_hbm, gam_hbm, bet_hbm, out_hbm,
          idx_all, rows, outs, pos_v, gam_v, bet_v, gsems, osems):
  bpw = ids_hbm.shape[0] // NW
  wid = lax.axis_index("s") * NC + lax.axis_index("c")
  base = wid * bpw

  pltpu.sync_copy(ids_hbm.at[pl.ds(base, bpw)], idx_all)
  pltpu.sync_copy(pos_hbm, pos_v)
  pltpu.sync_copy(gam_hbm, gam_v)
  pltpu.sync_copy(bet_hbm, bet_v)
  g = [gam_v[pl.ds(16 * j, 16)] for j in range(NV)]
  b = [bet_v[pl.ds(16 * j, 16)] for j in range(NV)]
  lanes = lax.iota(jnp.int32, 16)
  hidx = [(lanes + 16 * j) >> 3 for j in range(NV)]
  midx = [(lanes + 16 * j) & 7 for j in range(NV)]
  zidx = lanes & 0

  def oref(blk):
    return out_hbm.at[blk // 32, :, pl.ds(blk % 32, 1)]

  def compute_block(rv, ov, t, carry):
    p = [pos_v[t, pl.ds(16 * j, 16)] for j in range(NV)]

    @plsc.parallel_loop(0, BLK, unroll=2)
    def row_body(r):
      x = [rv[r, pl.ds(16 * j, 16)] + p[j] for j in range(NV)]
      s = (x[0] + x[1]) + (x[2] + x[3])
      q = (x[0] * x[0] + x[1] * x[1]) + (x[2] * x[2] + x[3] * x[3])
      mean = _lane_sum(s) * (1.0 / H)
      var = _lane_sum(q) * (1.0 / H) - mean * mean
      inv = _rsqrt(var + 1e-12)
      rsp = jnp.full((16,), r, dtype=jnp.int32)
      for j in range(NV):
        y = (x[j] - mean) * inv * g[j] + b[j]
        plsc.store_scatter(ov, [hidx[j], zidx, midx[j], rsp], y)
    return carry

  pltpu.async_copy(table_hbm.at[idx_all.at[0]], rows[0], gsems[0])
  pltpu.async_copy(table_hbm.at[idx_all.at[1]], rows[1], gsems[1])
  pltpu.async_copy(table_hbm.at[idx_all.at[2]], rows[2], gsems[2])

  def ring_body(it, carry):
    for bb in range(NBUF):
      c = it * NBUF + bb
      b2 = (bb + 3) % NBUF

      @pl.when(c + 3 < bpw)
      def _():
        pltpu.async_copy(
            table_hbm.at[idx_all.at[c + 3]], rows[b2], gsems[b2])

      pltpu.make_async_copy(
          table_hbm.at[idx_all.at[c]], rows[bb], gsems[bb]).wait()

      @pl.when(c >= NBUF)
      def _():
        pltpu.make_async_copy(outs[bb].at[:, :, :, pl.ds(0, BLK)],
                              oref(base + c - NBUF), osems[bb]).wait()

      compute_block(rows[bb], outs[bb], (base + c) // 32, 0)
      pltpu.async_copy(outs[bb].at[:, :, :, pl.ds(0, BLK)],
                       oref(base + c), osems[bb])
    return carry

  lax.fori_loop(0, bpw // NBUF, ring_body, 0)

  for bb in range(NBUF):
    pltpu.make_async_copy(
        outs[bb].at[:, :, :, pl.ds(0, BLK)],
        oref(base + bpw - NBUF + bb), osems[bb]).wait()


def kernel(input_ids, item_table, pos_table, ln_gamma, ln_beta):
  batch, seq = input_ids.shape
  nblk = batch * seq // BLK
  ids_t = (input_ids.T.astype(jnp.int32) * 2).reshape(nblk, BLK)
  item_table = jnp.pad(item_table, ((0, 0), (0, 64)))
  item_table = item_table.reshape(2 * item_table.shape[0] // 2, 2 * 64).reshape(-1, 64)

  def body(ids_hbm, table_hbm, pos_hbm, gam_hbm, bet_hbm, out_hbm,
           idx_all, r0, r1, r2, r3, o0, o1, o2, o3, pos_v, gam_v, bet_v,
           g0, g1, g2, g3, s0, s1, s2, s3):
    _body(ids_hbm, table_hbm, pos_hbm, gam_hbm, bet_hbm, out_hbm,
          idx_all, [r0, r1, r2, r3], [o0, o1, o2, o3], pos_v, gam_v, bet_v,
          [g0, g1, g2, g3], [s0, s1, s2, s3])

  mesh = plsc.VectorSubcoreMesh(core_axis_name="c", subcore_axis_name="s")
  run = pl.kernel(
      body,
      mesh=mesh,
      compiler_params=pltpu.CompilerParams(
          use_tc_tiling_on_sc=False, needs_layout_passes=False),
      out_type=jax.ShapeDtypeStruct((SEQ, 8, batch // BLK, 8, BLK),
                                    jnp.float32),
      scratch_types=(
          [pltpu.VMEM((nblk // NW, BLK), jnp.int32)]
          + [pltpu.VMEM((BLK, H), jnp.float32) for _ in range(NBUF)]
          + [pltpu.VMEM((8, 1, 8, BLK + 1), jnp.float32) for _ in range(NBUF)]
          + [pltpu.VMEM((SEQ, H), jnp.float32),
             pltpu.VMEM((H,), jnp.float32),
             pltpu.VMEM((H,), jnp.float32)]
          + [pltpu.SemaphoreType.DMA for _ in range(2 * NBUF)]
      ),
  )
  out5 = run(ids_t, item_table, pos_table, ln_gamma, ln_beta)
  return out5.transpose(2, 4, 0, 1, 3).reshape(batch, seq, H)
